# baseline (speedup 1.0000x reference)
import jax
import jax.numpy as jnp
from jax.experimental import pallas as pl
from jax.experimental.pallas import tpu as pltpu

ALPHA = 0.2
NEG_BIG = -9000000000000000.0
ROW_BLOCK = 256


def _proj_kernel(h_ref, w_ref, a1_ref, a2_ref, hw_ref, u_ref, v_ref):
    hw = jnp.dot(h_ref[...], w_ref[...], preferred_element_type=jnp.float32,
                 precision=jax.lax.Precision.HIGHEST)
    hw_ref[...] = hw
    u_ref[...] = jnp.dot(hw, a1_ref[...], preferred_element_type=jnp.float32,
                         precision=jax.lax.Precision.HIGHEST)
    v_ref[...] = jnp.dot(hw, a2_ref[...], preferred_element_type=jnp.float32,
                         precision=jax.lax.Precision.HIGHEST)


def _evalue_kernel(na_ref, u_ref, v_ref, ep_ref, em_ref):
    na = na_ref[...]
    u = u_ref[...]
    v = v_ref[...]
    gt = na > 0
    lt = na < 0
    zero = jnp.zeros_like(na)
    acc_p = jnp.sum(jnp.where(gt, u, zero) + jnp.where(lt, v, zero),
                    axis=1, keepdims=True)
    acc_m = jnp.sum(jnp.where(gt, v, zero) + jnp.where(lt, u, zero),
                    axis=1, keepdims=True)
    ep_ref[...] = jnp.where(acc_p >= 0, acc_p, ALPHA * acc_p)
    em_ref[...] = jnp.where(acc_m >= 0, acc_m, ALPHA * acc_m)


def _attn_kernel(ea_ref, ep_ref, em_ref, hw_ref, out_ref):
    ea = ea_ref[...]
    gt = ea > 0
    lt = ea < 0
    logits = jnp.where(gt, ep_ref[...], jnp.where(lt, em_ref[...], NEG_BIG))
    m = jnp.max(logits, axis=1, keepdims=True)
    p = jnp.exp(logits - m)
    s = jnp.sum(p, axis=1, keepdims=True)
    w = p * ea
    acc = jnp.dot(w, hw_ref[...], preferred_element_type=jnp.float32,
                  precision=jax.lax.Precision.HIGHEST)
    out_ref[...] = acc / s


def kernel(h, node_adj, edge_adj, W, a):
    n, in_f = h.shape
    out_f = W.shape[1]
    rb = ROW_BLOCK
    a1 = a[:out_f]
    a2 = a[out_f:]

    hw, u, v = pl.pallas_call(
        _proj_kernel,
        out_shape=[
            jax.ShapeDtypeStruct((n, out_f), jnp.float32),
            jax.ShapeDtypeStruct((n, 1), jnp.float32),
            jax.ShapeDtypeStruct((n, 1), jnp.float32),
        ],
    )(h, W, a1, a2)

    u_row = u.reshape(1, n)
    v_row = v.reshape(1, n)

    ep, em = pl.pallas_call(
        _evalue_kernel,
        grid=(n // rb,),
        in_specs=[
            pl.BlockSpec((rb, n), lambda i: (i, 0)),
            pl.BlockSpec((1, n), lambda i: (0, 0)),
            pl.BlockSpec((1, n), lambda i: (0, 0)),
        ],
        out_specs=[
            pl.BlockSpec((rb, 1), lambda i: (i, 0)),
            pl.BlockSpec((rb, 1), lambda i: (i, 0)),
        ],
        out_shape=[
            jax.ShapeDtypeStruct((n, 1), jnp.float32),
            jax.ShapeDtypeStruct((n, 1), jnp.float32),
        ],
        compiler_params=pltpu.CompilerParams(
            dimension_semantics=("parallel",)),
    )(node_adj, u_row, v_row)

    ep_row = ep.reshape(1, n)
    em_row = em.reshape(1, n)

    h_prime = pl.pallas_call(
        _attn_kernel,
        grid=(n // rb,),
        in_specs=[
            pl.BlockSpec((rb, n), lambda i: (i, 0)),
            pl.BlockSpec((1, n), lambda i: (0, 0)),
            pl.BlockSpec((1, n), lambda i: (0, 0)),
            pl.BlockSpec((n, out_f), lambda i: (0, 0)),
        ],
        out_specs=pl.BlockSpec((rb, out_f), lambda i: (i, 0)),
        out_shape=jax.ShapeDtypeStruct((n, out_f), jnp.float32),
        compiler_params=pltpu.CompilerParams(
            dimension_semantics=("parallel",)),
    )(edge_adj, ep_row, em_row, hw)

    return h_prime


# speedup vs baseline: 1.6196x; 1.6196x over previous
import jax
import jax.numpy as jnp
from jax.experimental import pallas as pl
from jax.experimental.pallas import tpu as pltpu

ALPHA = 0.2
NEG_BIG = -9000000000000000.0
ROW_BLOCK = 256


def _proj_kernel(h_ref, w_ref, hw_ref):
    hw_ref[...] = jnp.dot(h_ref[...], w_ref[...],
                          preferred_element_type=jnp.float32)


def _evalue_kernel(na_ref, hw_bf_ref, a_bf_ref, ep_ref, em_ref):
    na = na_ref[...]
    hw_bf = hw_bf_ref[...]
    a_bf = a_bf_ref[...]
    mp = (na > 0).astype(jnp.bfloat16)
    mm = (na < 0).astype(jnp.bfloat16)
    hp = jnp.dot(mp, hw_bf, preferred_element_type=jnp.float32)
    hm = jnp.dot(mm, hw_bf, preferred_element_type=jnp.float32)
    aip = jnp.concatenate([hp, hm], axis=1).astype(jnp.bfloat16)
    aim = jnp.concatenate([hm, hp], axis=1).astype(jnp.bfloat16)
    e_p = jnp.dot(aip, a_bf, preferred_element_type=jnp.float32)
    e_m = jnp.dot(aim, a_bf, preferred_element_type=jnp.float32)
    ep_ref[...] = jnp.where(e_p >= 0, e_p, ALPHA * e_p)
    em_ref[...] = jnp.where(e_m >= 0, e_m, ALPHA * e_m)


def _attn_kernel(ea_ref, ep_ref, em_ref, hw_bf_ref, out_ref):
    ea = ea_ref[...]
    gt = ea > 0
    lt = ea < 0
    logits = jnp.where(gt, ep_ref[...], jnp.where(lt, em_ref[...], NEG_BIG))
    m = jnp.max(logits, axis=1, keepdims=True)
    p = jnp.exp(logits - m)
    s = jnp.sum(p, axis=1, keepdims=True)
    w = ((p / s) * ea).astype(jnp.bfloat16)
    out_ref[...] = jnp.dot(w, hw_bf_ref[...],
                           preferred_element_type=jnp.float32)


def kernel(h, node_adj, edge_adj, W, a):
    n, in_f = h.shape
    out_f = W.shape[1]
    rb = ROW_BLOCK

    hw = pl.pallas_call(
        _proj_kernel,
        out_shape=jax.ShapeDtypeStruct((n, out_f), jnp.float32),
    )(h.astype(jnp.bfloat16), W.astype(jnp.bfloat16))

    hw_bf = hw.astype(jnp.bfloat16)
    a_bf = a.astype(jnp.bfloat16)

    ep, em = pl.pallas_call(
        _evalue_kernel,
        grid=(n // rb,),
        in_specs=[
            pl.BlockSpec((rb, n), lambda i: (i, 0)),
            pl.BlockSpec((n, out_f), lambda i: (0, 0)),
            pl.BlockSpec((2 * out_f, 1), lambda i: (0, 0)),
        ],
        out_specs=[
            pl.BlockSpec((rb, 1), lambda i: (i, 0)),
            pl.BlockSpec((rb, 1), lambda i: (i, 0)),
        ],
        out_shape=[
            jax.ShapeDtypeStruct((n, 1), jnp.float32),
            jax.ShapeDtypeStruct((n, 1), jnp.float32),
        ],
        compiler_params=pltpu.CompilerParams(
            dimension_semantics=("parallel",)),
    )(node_adj, hw_bf, a_bf)

    ep_row = ep.reshape(1, n)
    em_row = em.reshape(1, n)

    h_prime = pl.pallas_call(
        _attn_kernel,
        grid=(n // rb,),
        in_specs=[
            pl.BlockSpec((rb, n), lambda i: (i, 0)),
            pl.BlockSpec((1, n), lambda i: (0, 0)),
            pl.BlockSpec((1, n), lambda i: (0, 0)),
            pl.BlockSpec((n, out_f), lambda i: (0, 0)),
        ],
        out_specs=pl.BlockSpec((rb, out_f), lambda i: (i, 0)),
        out_shape=jax.ShapeDtypeStruct((n, out_f), jnp.float32),
        compiler_params=pltpu.CompilerParams(
            dimension_semantics=("parallel",)),
    )(edge_adj, ep_row, em_row, hw_bf)

    return h_prime


# speedup vs baseline: 1.7565x; 1.0845x over previous
import jax
import jax.numpy as jnp
from jax.experimental import pallas as pl
from jax.experimental.pallas import tpu as pltpu

ALPHA = 0.2
NEG_BIG = -9000000000000000.0
ROW_BLOCK = 512


def _proj_kernel(h_ref, w_ref, hw_ref):
    hw_ref[...] = jnp.dot(h_ref[...], w_ref[...],
                          preferred_element_type=jnp.float32)


def _evalue_kernel(na_ref, hw_bf_ref, a_bf_ref, ep_ref, em_ref):
    na = na_ref[...]
    hw_bf = hw_bf_ref[...]
    a_bf = a_bf_ref[...]
    mp = (na > 0).astype(jnp.bfloat16)
    mm = (na < 0).astype(jnp.bfloat16)
    hp = jnp.dot(mp, hw_bf, preferred_element_type=jnp.float32)
    hm = jnp.dot(mm, hw_bf, preferred_element_type=jnp.float32)
    aip = jnp.concatenate([hp, hm], axis=1).astype(jnp.bfloat16)
    aim = jnp.concatenate([hm, hp], axis=1).astype(jnp.bfloat16)
    e_p = jnp.dot(aip, a_bf, preferred_element_type=jnp.float32)
    e_m = jnp.dot(aim, a_bf, preferred_element_type=jnp.float32)
    ep_ref[...] = jnp.where(e_p >= 0, e_p, ALPHA * e_p)
    em_ref[...] = jnp.where(e_m >= 0, e_m, ALPHA * e_m)


def _attn_kernel(ea_ref, ep_ref, em_ref, hw_bf_ref, out_ref):
    ea = ea_ref[...]
    gt = ea > 0
    lt = ea < 0
    logits = jnp.where(gt, ep_ref[...], jnp.where(lt, em_ref[...], NEG_BIG))
    m = jnp.max(logits, axis=1, keepdims=True)
    p = jnp.exp(logits - m)
    s = jnp.sum(p, axis=1, keepdims=True)
    w = ((p / s) * ea).astype(jnp.bfloat16)
    out_ref[...] = jnp.dot(w, hw_bf_ref[...],
                           preferred_element_type=jnp.float32)


def kernel(h, node_adj, edge_adj, W, a):
    n, in_f = h.shape
    out_f = W.shape[1]
    rb = ROW_BLOCK

    hw = pl.pallas_call(
        _proj_kernel,
        out_shape=jax.ShapeDtypeStruct((n, out_f), jnp.float32),
    )(h.astype(jnp.bfloat16), W.astype(jnp.bfloat16))

    hw_bf = hw.astype(jnp.bfloat16)
    a_bf = a.astype(jnp.bfloat16)

    ep, em = pl.pallas_call(
        _evalue_kernel,
        grid=(n // rb,),
        in_specs=[
            pl.BlockSpec((rb, n), lambda i: (i, 0)),
            pl.BlockSpec((n, out_f), lambda i: (0, 0)),
            pl.BlockSpec((2 * out_f, 1), lambda i: (0, 0)),
        ],
        out_specs=[
            pl.BlockSpec((rb, 1), lambda i: (i, 0)),
            pl.BlockSpec((rb, 1), lambda i: (i, 0)),
        ],
        out_shape=[
            jax.ShapeDtypeStruct((n, 1), jnp.float32),
            jax.ShapeDtypeStruct((n, 1), jnp.float32),
        ],
        compiler_params=pltpu.CompilerParams(
            dimension_semantics=("parallel",)),
    )(node_adj, hw_bf, a_bf)

    ep_row = ep.reshape(1, n)
    em_row = em.reshape(1, n)

    h_prime = pl.pallas_call(
        _attn_kernel,
        grid=(n // rb,),
        in_specs=[
            pl.BlockSpec((rb, n), lambda i: (i, 0)),
            pl.BlockSpec((1, n), lambda i: (0, 0)),
            pl.BlockSpec((1, n), lambda i: (0, 0)),
            pl.BlockSpec((n, out_f), lambda i: (0, 0)),
        ],
        out_specs=pl.BlockSpec((rb, out_f), lambda i: (i, 0)),
        out_shape=jax.ShapeDtypeStruct((n, out_f), jnp.float32),
        compiler_params=pltpu.CompilerParams(
            dimension_semantics=("parallel",)),
    )(edge_adj, ep_row, em_row, hw_bf)

    return h_prime
